# Initial kernel scaffold; baseline (speedup 1.0000x reference)
import jax
import jax.numpy as jnp
from jax.experimental import pallas as pl


def kernel(inputs):
    raise NotImplementedError("write your pallas kernel here")


# speedup vs baseline: 1.1134x; 1.1134x over previous
import jax
import jax.numpy as jnp
from jax.experimental import pallas as pl
from jax.experimental.pallas import tpu as pltpu


def _extract_body(x_hbm_ref, o_ref, sem):
    copy = pltpu.make_async_copy(x_hbm_ref.at[:, 0, :], o_ref, sem)
    copy.start()
    copy.wait()


def kernel(inputs):
    B, S, D = inputs.shape
    return pl.pallas_call(
        _extract_body,
        in_specs=[pl.BlockSpec(memory_space=pl.ANY)],
        out_specs=pl.BlockSpec((B, D), lambda: (0, 0)),
        out_shape=jax.ShapeDtypeStruct((B, D), inputs.dtype),
        scratch_shapes=[pltpu.SemaphoreType.DMA],
    )(inputs)
